# baseline (speedup 1.0000x reference)
import math

import jax
import jax.numpy as jnp
from jax.experimental import pallas as pl
from jax.experimental.pallas import tpu as pltpu

_B, _S, _H, _E = 4, 2048, 1024, 16
_ROWS = _B * _S
_CHUNK = 512
_NSTEPS = _ROWS // _CHUNK
_CHUNKS_PER_BATCH = _S // _CHUNK
_INV_SQRT2 = 1.0 / math.sqrt(2.0)


def _gate_kernel(x_ref, w1_ref, b1_ref, w2_ref, b2_ref, tw_ref, ti_ref, acc_ref):
    step = pl.program_id(0)

    @pl.when(step == 0)
    def _init():
        acc_ref[...] = jnp.zeros_like(acc_ref)

    rows = jax.lax.broadcasted_iota(jnp.int32, (8, _CHUNK), 0)
    sel = (rows == step // _CHUNKS_PER_BATCH).astype(jnp.float32)
    acc_ref[...] += jnp.dot(sel, x_ref[...], preferred_element_type=jnp.float32)

    @pl.when(step == _NSTEPS - 1)
    def _tail():
        seq = acc_ref[...] * (1.0 / _S)
        h = jnp.dot(seq, w1_ref[...], preferred_element_type=jnp.float32)
        h = h + b1_ref[...]
        h = 0.5 * h * (1.0 + jax.lax.erf(h * _INV_SQRT2))
        logits = jnp.dot(h, w2_ref[...], preferred_element_type=jnp.float32)
        logits = logits + b2_ref[...]
        m = jnp.max(logits, axis=1, keepdims=True)
        ex = jnp.exp(logits - m)
        probs = ex / jnp.sum(ex, axis=1, keepdims=True)
        lane = jax.lax.broadcasted_iota(jnp.int32, probs.shape, 1)
        p1 = jnp.max(probs, axis=1, keepdims=True)
        i1 = jnp.min(jnp.where(probs == p1, lane, _E), axis=1, keepdims=True)
        masked = jnp.where(lane == i1, -1.0, probs)
        p2 = jnp.max(masked, axis=1, keepdims=True)
        i2 = jnp.min(jnp.where(masked == p2, lane, _E), axis=1, keepdims=True)
        e2 = jnp.exp(p2 - p1)
        denom = 1.0 + e2
        tw = jnp.concatenate([1.0 / denom, e2 / denom], axis=1)
        ti = jnp.concatenate([i1, i2], axis=1)
        tw_ref[...] = tw[0:_B, :]
        ti_ref[...] = ti[0:_B, :]


def kernel(hidden_states, W1, b1, W2, b2):
    hs2 = hidden_states.reshape(_ROWS, _H)
    tw, ti = pl.pallas_call(
        _gate_kernel,
        grid=(_NSTEPS,),
        in_specs=[
            pl.BlockSpec((_CHUNK, _H), lambda i: (i, 0)),
            pl.BlockSpec((_H, _H), lambda i: (0, 0)),
            pl.BlockSpec((_H,), lambda i: (0,)),
            pl.BlockSpec((_H, _E), lambda i: (0, 0)),
            pl.BlockSpec((_E,), lambda i: (0,)),
        ],
        out_specs=[
            pl.BlockSpec((_B, 2), lambda i: (0, 0)),
            pl.BlockSpec((_B, 2), lambda i: (0, 0)),
        ],
        out_shape=[
            jax.ShapeDtypeStruct((_B, 2), jnp.float32),
            jax.ShapeDtypeStruct((_B, 2), jnp.int32),
        ],
        scratch_shapes=[pltpu.VMEM((8, _H), jnp.float32)],
    )(hs2, W1, b1, W2, b2)
    return tw, ti


# speedup vs baseline: 1.1925x; 1.1925x over previous
import math

import jax
import jax.numpy as jnp
from jax.experimental import pallas as pl
from jax.experimental.pallas import tpu as pltpu

_B, _S, _H, _E = 4, 2048, 1024, 16
_ROWS = _B * _S
_CHUNK = 1024
_FOLD = 256
_NSTEPS = _ROWS // _CHUNK
_CHUNKS_PER_BATCH = _S // _CHUNK
_INV_SQRT2 = 1.0 / math.sqrt(2.0)


def _gate_kernel(x_ref, w1_ref, b1_ref, w2_ref, b2_ref, tw_ref, ti_ref, acc_ref):
    step = pl.program_id(0)
    b = step // _CHUNKS_PER_BATCH

    partial = (x_ref[0:256] + x_ref[256:512]) + (x_ref[512:768] + x_ref[768:1024])

    @pl.when(step % _CHUNKS_PER_BATCH == 0)
    def _first_chunk_of_batch():
        acc_ref[pl.ds(b, 1)] = partial[None]

    @pl.when(step % _CHUNKS_PER_BATCH != 0)
    def _later_chunk_of_batch():
        acc_ref[pl.ds(b, 1)] += partial[None]

    @pl.when(step == _NSTEPS - 1)
    def _tail():
        seq = jnp.sum(acc_ref[...], axis=1) * (1.0 / _S)
        h = jnp.dot(seq, w1_ref[...], preferred_element_type=jnp.float32)
        h = h + b1_ref[...]
        h = 0.5 * h * (1.0 + jax.lax.erf(h * _INV_SQRT2))
        logits = jnp.dot(h, w2_ref[...], preferred_element_type=jnp.float32)
        logits = logits + b2_ref[...]
        m = jnp.max(logits, axis=1, keepdims=True)
        ex = jnp.exp(logits - m)
        probs = ex / jnp.sum(ex, axis=1, keepdims=True)
        lane = jax.lax.broadcasted_iota(jnp.int32, probs.shape, 1)
        p1 = jnp.max(probs, axis=1, keepdims=True)
        i1 = jnp.min(jnp.where(probs == p1, lane, _E), axis=1, keepdims=True)
        masked = jnp.where(lane == i1, -1.0, probs)
        p2 = jnp.max(masked, axis=1, keepdims=True)
        i2 = jnp.min(jnp.where(masked == p2, lane, _E), axis=1, keepdims=True)
        e2 = jnp.exp(p2 - p1)
        denom = 1.0 + e2
        tw_ref[...] = jnp.concatenate([1.0 / denom, e2 / denom], axis=1)
        ti_ref[...] = jnp.concatenate([i1, i2], axis=1)


def kernel(hidden_states, W1, b1, W2, b2):
    hs2 = hidden_states.reshape(_ROWS, _H)
    tw, ti = pl.pallas_call(
        _gate_kernel,
        grid=(_NSTEPS,),
        in_specs=[
            pl.BlockSpec((_CHUNK, _H), lambda i: (i, 0)),
            pl.BlockSpec((_H, _H), lambda i: (0, 0)),
            pl.BlockSpec((_H,), lambda i: (0,)),
            pl.BlockSpec((_H, _E), lambda i: (0, 0)),
            pl.BlockSpec((_E,), lambda i: (0,)),
        ],
        out_specs=[
            pl.BlockSpec((_B, 2), lambda i: (0, 0)),
            pl.BlockSpec((_B, 2), lambda i: (0, 0)),
        ],
        out_shape=[
            jax.ShapeDtypeStruct((_B, 2), jnp.float32),
            jax.ShapeDtypeStruct((_B, 2), jnp.int32),
        ],
        scratch_shapes=[pltpu.VMEM((_B, _FOLD, _H), jnp.float32)],
    )(hs2, W1, b1, W2, b2)
    return tw, ti


# speedup vs baseline: 1.2200x; 1.0230x over previous
import math

import jax
import jax.numpy as jnp
from jax.experimental import pallas as pl
from jax.experimental.pallas import tpu as pltpu

_B, _S, _H, _E = 4, 2048, 1024, 16
_ROWS = _B * _S
_CHUNK = 512
_FOLD = 256
_NSTEPS = _S // _CHUNK
_INV_SQRT2 = 1.0 / math.sqrt(2.0)


def _gate_kernel(x0_ref, x1_ref, x2_ref, x3_ref, w1_ref, b1_ref, w2_ref, b2_ref,
                 tw_ref, ti_ref, a0_ref, a1_ref, a2_ref, a3_ref):
    step = pl.program_id(0)
    xs = (x0_ref, x1_ref, x2_ref, x3_ref)
    accs = (a0_ref, a1_ref, a2_ref, a3_ref)
    partials = [x[0:256] + x[256:512] for x in xs]

    @pl.when(step == 0)
    def _init():
        for acc, p in zip(accs, partials):
            acc[...] = p

    @pl.when(step != 0)
    def _accum():
        for acc, p in zip(accs, partials):
            acc[...] += p

    @pl.when(step == _NSTEPS - 1)
    def _tail():
        rows = [jnp.sum(acc[...], axis=0, keepdims=True) for acc in accs]
        seq = jnp.concatenate(rows, axis=0) * (1.0 / _S)
        h = jnp.dot(seq, w1_ref[...], preferred_element_type=jnp.float32)
        h = h + b1_ref[...]
        h = 0.5 * h * (1.0 + jax.lax.erf(h * _INV_SQRT2))
        logits = jnp.dot(h, w2_ref[...], preferred_element_type=jnp.float32)
        logits = logits + b2_ref[...]
        m = jnp.max(logits, axis=1, keepdims=True)
        ex = jnp.exp(logits - m)
        probs = ex / jnp.sum(ex, axis=1, keepdims=True)
        lane = jax.lax.broadcasted_iota(jnp.int32, probs.shape, 1)
        p1 = jnp.max(probs, axis=1, keepdims=True)
        i1 = jnp.min(jnp.where(probs == p1, lane, _E), axis=1, keepdims=True)
        masked = jnp.where(lane == i1, -1.0, probs)
        p2 = jnp.max(masked, axis=1, keepdims=True)
        i2 = jnp.min(jnp.where(masked == p2, lane, _E), axis=1, keepdims=True)
        e2 = jnp.exp(p2 - p1)
        denom = 1.0 + e2
        tw_ref[...] = jnp.concatenate([1.0 / denom, e2 / denom], axis=1)
        ti_ref[...] = jnp.concatenate([i1, i2], axis=1)


def _x_spec(quarter):
    blocks_per_quarter = _S // _CHUNK
    return pl.BlockSpec((_CHUNK, _H),
                        lambda i, q=quarter: (q * blocks_per_quarter + i, 0))


def kernel(hidden_states, W1, b1, W2, b2):
    hs2 = hidden_states.reshape(_ROWS, _H)
    tw, ti = pl.pallas_call(
        _gate_kernel,
        grid=(_NSTEPS,),
        in_specs=[
            _x_spec(0),
            _x_spec(1),
            _x_spec(2),
            _x_spec(3),
            pl.BlockSpec((_H, _H), lambda i: (0, 0)),
            pl.BlockSpec((_H,), lambda i: (0,)),
            pl.BlockSpec((_H, _E), lambda i: (0, 0)),
            pl.BlockSpec((_E,), lambda i: (0,)),
        ],
        out_specs=[
            pl.BlockSpec((_B, 2), lambda i: (0, 0)),
            pl.BlockSpec((_B, 2), lambda i: (0, 0)),
        ],
        out_shape=[
            jax.ShapeDtypeStruct((_B, 2), jnp.float32),
            jax.ShapeDtypeStruct((_B, 2), jnp.int32),
        ],
        scratch_shapes=[pltpu.VMEM((_FOLD, _H), jnp.float32) for _ in range(4)],
    )(hs2, hs2, hs2, hs2, W1, b1, W2, b2)
    return tw, ti
